# Initial kernel scaffold; baseline (speedup 1.0000x reference)
import jax
import jax.numpy as jnp
from jax.experimental import pallas as pl


def kernel(feature, label):
    raise NotImplementedError("write your pallas kernel here")


# speedup vs baseline: 19.0093x; 19.0093x over previous
import functools

import jax
import jax.numpy as jnp
import numpy as np
from jax.experimental import pallas as pl
from jax.experimental.pallas import tpu as pltpu

_NUM_CLASSES = 19
_P = 96 * 96
_C = 128
_KS = np.arange(-3, 4).astype(np.float32)
_TW = np.exp(-0.5 * _KS**2)
_TARGET = (_TW / _TW.sum()).reshape(1, 7)


def _loss_kernel(x_ref, labt_ref, out_ref):
    x = x_ref[:]
    lab = labt_ref[:]

    rows = [
        jnp.sum(jnp.where(lab == c, 1.0, 0.0), axis=0, keepdims=True)
        for c in range(_NUM_CLASSES)
    ]
    cnt = jnp.concatenate(rows, axis=0)

    dn = (((1,), (1,)), ((), ()))
    s1 = jax.lax.dot_general(x, cnt, dn, precision=jax.lax.Precision.HIGHEST,
                             preferred_element_type=jnp.float32)
    s2 = jax.lax.dot_general(x * x, cnt, dn,
                             precision=jax.lax.Precision.HIGHEST,
                             preferred_element_type=jnp.float32)

    kvec = jax.lax.broadcasted_iota(jnp.int32, (1, 7), 1).astype(jnp.float32) - 3.0
    tw = jnp.exp(-0.5 * kvec * kvec)
    target = tw / jnp.sum(tw)
    loss_acc = jnp.float32(0.0)
    act_acc = jnp.float32(0.0)
    for c in range(_NUM_CLASSES):
        cp = cnt[c:c + 1, :]
        n_c = jnp.sum(cp)
        nsafe = jnp.maximum(n_c, 1.0)
        mu = s1[:, c:c + 1] / nsafe
        e2 = s2[:, c:c + 1] / nsafe
        var = e2 - mu * mu * (2.0 - n_c / nsafe) + 1e-10
        inv_std = jax.lax.rsqrt(var)
        z = (mu - x) * inv_std
        us = []
        for k in range(-3, 4):
            zk = z + jnp.float32(k)
            e = jnp.exp(-12.5 * zk * zk)
            us.append(jax.lax.dot_general(
                e, cp, dn, precision=jax.lax.Precision.HIGHEST,
                preferred_element_type=jnp.float32))
        u = jnp.concatenate(us, axis=1)
        ssum = jnp.sum(u, axis=1, keepdims=True)
        hist = u / ssum
        d = jnp.abs(hist - target)
        sl = jnp.where(d < 1.0, 0.5 * d * d, d - 0.5)
        lc = jnp.sum(sl) * jnp.float32(1.0 / (_C * 7))
        active = n_c >= 1000.0
        loss_acc = loss_acc + jnp.where(active, lc, 0.0)
        act_acc = act_acc + jnp.where(active, 1.0, 0.0)

    out_ref[0, 0] = loss_acc / act_acc


@functools.partial(jax.jit)
def kernel(feature, label):
    x = feature[0].reshape(_C, _P)
    labt = (label[0, 0].astype(jnp.int32)
            .reshape(96, 4, 96, 4).transpose(1, 3, 0, 2).reshape(16, _P))
    out = pl.pallas_call(
        _loss_kernel,
        out_shape=jax.ShapeDtypeStruct((1, 1), jnp.float32),
        out_specs=pl.BlockSpec(memory_space=pltpu.SMEM),
    )(x, labt)
    return out[0, 0]
